# Initial kernel scaffold; baseline (speedup 1.0000x reference)
import jax
import jax.numpy as jnp
from jax.experimental import pallas as pl


def kernel(features_0, features_1, features_2, e_feat, edge_index, W_fc0, b_fc0, W_fc1, b_fc1, W_fc2, b_fc2, W1, W2):
    raise NotImplementedError("write your pallas kernel here")


# speedup vs baseline: 4.5640x; 4.5640x over previous
import functools

import jax
import jax.numpy as jnp
from jax import lax
from jax.experimental import pallas as pl
from jax.experimental.pallas import tpu as pltpu
from jax.experimental.pallas import tpu_sc as plsc

NA, NB, NC_T = 40000, 30000, 30000
NN = NA + NB + NC_T
EE = 1600000
NTS = 3
HH = 16
FW = NTS * HH
CO = 8

SC_CORES, SC_SUBCORES = 2, 16
NW = SC_CORES * SC_SUBCORES
CH = 128
NCHUNK = 392
GCH = 8
NGRP = NCHUNK // GCH
EPT = CH * NCHUNK
E_PAD = EPT * NW
N_PAD = 100352
STRIDE = N_PAD // SC_SUBCORES
SLABW = 8
NSLAB = FW // SLABW
DW = 8

_MESH = dict(core_axis_name="c", subcore_axis_name="s")


def _zero_stripe(zeros_v, acc, row0):
    def zbody(i, _):
        pltpu.sync_copy(zeros_v, acc.at[pl.ds(row0 + i * CH, CH)])
        return 0
    lax.fori_loop(0, STRIDE // CH, zbody, 0)


@functools.partial(
    pl.kernel,
    mesh=plsc.VectorSubcoreMesh(**_MESH),
    out_type=jax.ShapeDtypeStruct((2, SC_CORES, N_PAD, DW), jnp.float32),
    scratch_types=[
        pltpu.VMEM((GCH, CH), jnp.int32),
        pltpu.VMEM((CH, DW), jnp.float32),
        pltpu.VMEM((CH, DW), jnp.float32),
        pltpu.VMEM_SHARED((N_PAD, DW), jnp.float32),
    ],
    compiler_params=pltpu.CompilerParams(use_tc_tiling_on_sc=False),
)
def _deg_kernel(src_hbm, dst_hbm, ones_hbm, zeros_hbm, out_hbm,
                idx_v, ones_v, zeros_v, acc):
    c = lax.axis_index("c")
    s = lax.axis_index("s")
    wid = s * SC_CORES + c
    row0 = s * STRIDE
    pltpu.sync_copy(ones_hbm, ones_v)
    pltpu.sync_copy(zeros_hbm, zeros_v)
    for io in range(2):
        e_hbm = dst_hbm if io == 0 else src_hbm
        _zero_stripe(zeros_v, acc, row0)
        plsc.subcore_barrier()

        def gbody(g, _):
            pltpu.sync_copy(e_hbm.at[pl.ds(wid * NCHUNK + g * GCH, GCH)],
                            idx_v)

            def ebody(j, _):
                pltpu.sync_copy(ones_v, acc.at[idx_v.at[j]], add=True)
                return 0
            lax.fori_loop(0, GCH, ebody, 0)
            return 0
        lax.fori_loop(0, NGRP, gbody, 0)
        plsc.subcore_barrier()
        pltpu.sync_copy(acc.at[pl.ds(row0, STRIDE)],
                        out_hbm.at[io, c, pl.ds(row0, STRIDE)])


@functools.partial(
    pl.kernel,
    mesh=plsc.VectorSubcoreMesh(**_MESH),
    out_type=jax.ShapeDtypeStruct((NSLAB, SC_CORES, N_PAD, SLABW),
                                  jnp.float32),
    scratch_types=[
        pltpu.VMEM((GCH, CH), jnp.int32),
        pltpu.VMEM((GCH, CH), jnp.int32),
        pltpu.VMEM((CH, SLABW), jnp.float32),
        pltpu.VMEM((CH, SLABW), jnp.float32),
        pltpu.VMEM((CH, SLABW), jnp.float32),
        pltpu.VMEM_SHARED((N_PAD, SLABW), jnp.float32),
        pltpu.SemaphoreType.DMA,
    ],
    compiler_params=pltpu.CompilerParams(use_tc_tiling_on_sc=False),
)
def _prop_kernel(tab0_hbm, tab1_hbm, tab2_hbm, tab3_hbm, tab4_hbm, tab5_hbm,
                 src_hbm, dst_hbm, zeros_hbm, out_hbm, sidx_v, didx_v, rows_a,
                 rows_b, zeros_v, acc, sem):
    c = lax.axis_index("c")
    s = lax.axis_index("s")
    wid = s * SC_CORES + c
    row0 = s * STRIDE
    pltpu.sync_copy(zeros_hbm, zeros_v)
    for t, tab in enumerate((tab0_hbm, tab1_hbm, tab2_hbm, tab3_hbm,
                             tab4_hbm, tab5_hbm)):
        _zero_stripe(zeros_v, acc, row0)
        plsc.subcore_barrier()

        def gbody(g, _):
            base = wid * NCHUNK + g * GCH
            pltpu.sync_copy(src_hbm.at[pl.ds(base, GCH)], sidx_v)
            pltpu.sync_copy(dst_hbm.at[pl.ds(base, GCH)], didx_v)
            pltpu.async_copy(tab.at[sidx_v.at[0]], rows_a, sem)

            def pbody(jj, _):
                j0 = 2 * jj
                pltpu.make_async_copy(tab.at[sidx_v.at[j0]], rows_a,
                                      sem).wait()
                pltpu.async_copy(tab.at[sidx_v.at[j0 + 1]], rows_b, sem)
                pltpu.sync_copy(rows_a, acc.at[didx_v.at[j0]], add=True)
                pltpu.make_async_copy(tab.at[sidx_v.at[j0 + 1]], rows_b,
                                      sem).wait()

                @pl.when(j0 + 2 < GCH)
                def _():
                    pltpu.async_copy(tab.at[sidx_v.at[j0 + 2]], rows_a, sem)
                pltpu.sync_copy(rows_b, acc.at[didx_v.at[j0 + 1]], add=True)
                return 0
            lax.fori_loop(0, GCH // 2, pbody, 0)
            return 0
        lax.fori_loop(0, NGRP, gbody, 0)
        plsc.subcore_barrier()
        pltpu.sync_copy(acc.at[pl.ds(row0, STRIDE)],
                        out_hbm.at[t, c, pl.ds(row0, STRIDE)])


def _norm_body(d_ref, o_ref):
    d = d_ref[0, 0] + d_ref[0, 1]
    o_ref[0] = lax.rsqrt(jnp.maximum(d, 1.0))


def _norms(deg):
    nrow = N_PAD * DW // 128
    d = deg.reshape(2, SC_CORES, nrow, 128)
    return pl.pallas_call(
        _norm_body,
        grid=(2, nrow // 128),
        in_specs=[pl.BlockSpec((1, SC_CORES, 128, 128),
                               lambda io, r: (io, 0, r, 0))],
        out_specs=pl.BlockSpec((1, 128, 128), lambda io, r: (io, r, 0)),
        out_shape=jax.ShapeDtypeStruct((2, nrow, 128), jnp.float32),
    )(d).reshape(2, N_PAD, DW)


def _enc_body(x_ref, w_ref, b_ref, n_ref, o_ref):
    y = lax.dot_general(x_ref[...], w_ref[...], (((1,), (0,)), ((), ())),
                        preferred_element_type=jnp.float32)
    o_ref[...] = (y + b_ref[...]) * n_ref[...]


def _encode(feats, w, b, nout16, row_off):
    nt = feats.shape[0]
    bt = 1000
    return pl.pallas_call(
        _enc_body,
        grid=(nt // bt,),
        in_specs=[
            pl.BlockSpec((bt, 128), lambda r: (r, 0)),
            pl.BlockSpec((128, HH), lambda r: (0, 0)),
            pl.BlockSpec((1, HH), lambda r: (0, 0)),
            pl.BlockSpec((bt, HH), lambda r: (r + row_off // bt, 0)),
        ],
        out_specs=pl.BlockSpec((bt, HH), lambda r: (r, 0)),
        out_shape=jax.ShapeDtypeStruct((nt, HH), jnp.float32),
    )(feats, w, b.reshape(1, HH), nout16)


def _combine(p_ref, ni_ref):
    m = jnp.concatenate([p_ref[k, 0] + p_ref[k, 1] for k in range(NSLAB)],
                        axis=1)
    return m * ni_ref[...]


def _stage1_body(p_ref, ni_ref, no_ref, o_ref):
    o_ref[...] = jnp.maximum(_combine(p_ref, ni_ref), 0.0) * no_ref[...]


def _stage2_body(p_ref, ni_ref, no_ref, w_ref, enc_ref, tab_ref):
    m = _combine(p_ref, ni_ref)
    hs = []
    for t in range(NTS):
        hs.append(lax.dot_general(m[:, t * HH:(t + 1) * HH], w_ref[t],
                                  (((1,), (0,)), ((), ())),
                                  preferred_element_type=jnp.float32))
    h = jnp.maximum(jnp.concatenate(hs, axis=1), 0.0)
    enc_ref[...] = h
    tab_ref[...] = h * no_ref[...]


def _stage3_body(p_ref, ni_ref, w_ref, o_ref):
    m = _combine(p_ref, ni_ref)
    acc = jnp.zeros((m.shape[0], CO), jnp.float32)
    for t in range(NTS):
        acc = acc + lax.dot_general(m[:, t * HH:(t + 1) * HH], w_ref[t],
                                    (((1,), (0,)), ((), ())),
                                    preferred_element_type=jnp.float32)
    o_ref[...] = acc * (1.0 / NTS)


_BT = 1024


def _pspec():
    return pl.BlockSpec((NSLAB, SC_CORES, _BT, SLABW), lambda r: (0, 0, r, 0))


def _nspec():
    return pl.BlockSpec((_BT, FW), lambda r: (r, 0))


def _stage1(p, nin48, nout48):
    return pl.pallas_call(
        _stage1_body,
        grid=(N_PAD // _BT,),
        in_specs=[_pspec(), _nspec(), _nspec()],
        out_specs=pl.BlockSpec((_BT, FW), lambda r: (r, 0)),
        out_shape=jax.ShapeDtypeStruct((N_PAD, FW), jnp.float32),
    )(p, nin48, nout48)


def _stage2(p, nin48, nout48, w1):
    return pl.pallas_call(
        _stage2_body,
        grid=(N_PAD // _BT,),
        in_specs=[_pspec(), _nspec(), _nspec(),
                  pl.BlockSpec((NTS, HH, HH), lambda r: (0, 0, 0))],
        out_specs=[pl.BlockSpec((_BT, FW), lambda r: (r, 0)),
                   pl.BlockSpec((_BT, FW), lambda r: (r, 0))],
        out_shape=[jax.ShapeDtypeStruct((N_PAD, FW), jnp.float32),
                   jax.ShapeDtypeStruct((N_PAD, FW), jnp.float32)],
    )(p, nin48, nout48, w1)


def _stage3(p, nin48, w2):
    return pl.pallas_call(
        _stage3_body,
        grid=(N_PAD // _BT,),
        in_specs=[_pspec(), _nspec(),
                  pl.BlockSpec((NTS, HH, CO), lambda r: (0, 0, 0))],
        out_specs=pl.BlockSpec((_BT, CO), lambda r: (r, 0)),
        out_shape=jax.ShapeDtypeStruct((N_PAD, CO), jnp.float32),
    )(p, nin48, w2)


def _slabs(tab48):
    return tuple(tab48[:, k * SLABW:(k + 1) * SLABW] for k in range(NSLAB))


def kernel(features_0, features_1, features_2, e_feat, edge_index,
           W_fc0, b_fc0, W_fc1, b_fc1, W_fc2, b_fc2, W1, W2):
    pad = jnp.full((E_PAD - EE,), NN, jnp.int32)
    src2d = jnp.concatenate([edge_index[0], pad]).reshape(E_PAD // CH, CH)
    dst2d = jnp.concatenate([edge_index[1], pad]).reshape(E_PAD // CH, CH)
    ones_blk = jnp.ones((CH, DW), jnp.float32)
    zeros_blk8 = jnp.zeros((CH, DW), jnp.float32)
    zeros_blk = jnp.zeros((CH, SLABW), jnp.float32)

    deg = _deg_kernel(src2d, dst2d, ones_blk, zeros_blk8)
    norms = _norms(deg)
    nin48 = jnp.broadcast_to(norms[0, :, 0:1], (N_PAD, FW))
    nout48 = jnp.broadcast_to(norms[1, :, 0:1], (N_PAD, FW))
    nout16 = nout48[:, :HH]

    enc0 = _encode(features_0, W_fc0, b_fc0, nout16, 0)
    enc1 = _encode(features_1, W_fc1, b_fc1, nout16, NA)
    enc2 = _encode(features_2, W_fc2, b_fc2, nout16, NA + NB)
    h0 = jnp.zeros((N_PAD, FW), jnp.float32)
    h0 = h0.at[0:NA, 0:HH].set(enc0)
    h0 = h0.at[NA:NA + NB, HH:2 * HH].set(enc1)
    h0 = h0.at[NA + NB:NN, 2 * HH:3 * HH].set(enc2)

    p1 = _prop_kernel(*_slabs(h0), src2d, dst2d, zeros_blk)
    tab1 = _stage1(p1, nin48, nout48)

    p2 = _prop_kernel(*_slabs(tab1), src2d, dst2d, zeros_blk)
    enc48, tab2 = _stage2(p2, nin48, nout48, W1)

    p3 = _prop_kernel(*_slabs(tab2), src2d, dst2d, zeros_blk)
    logits = _stage3(p3, nin48, W2)

    return (logits[:NN], enc48[:NN])


# speedup vs baseline: 7.7148x; 1.6904x over previous
import functools

import jax
import jax.numpy as jnp
from jax import lax
from jax.experimental import pallas as pl
from jax.experimental.pallas import tpu as pltpu
from jax.experimental.pallas import tpu_sc as plsc

NA, NB, NC_T = 40000, 30000, 30000
NN = NA + NB + NC_T
EE = 1600000
NTS = 3
HH = 16
FW = NTS * HH
CO = 8

SC_CORES, SC_SUBCORES = 2, 16
NW = SC_CORES * SC_SUBCORES
CH = 128
NCHUNK = 392
GCH = 56
NGRP = NCHUNK // GCH
NBUF = 8
EPT = CH * NCHUNK
E_PAD = EPT * NW
N_PAD = 100352
STRIDE = N_PAD // SC_SUBCORES
SLABW = 8
NSLAB = FW // SLABW
DW = 8

_MESH = dict(core_axis_name="c", subcore_axis_name="s")


def _zero_stripe(zeros_v, acc, row0):
    def zbody(i, _):
        pltpu.sync_copy(zeros_v, acc.at[pl.ds(row0 + i * CH, CH)])
        return 0
    lax.fori_loop(0, STRIDE // CH, zbody, 0)


@functools.partial(
    pl.kernel,
    mesh=plsc.VectorSubcoreMesh(**_MESH),
    out_type=jax.ShapeDtypeStruct((2, SC_CORES, N_PAD, DW), jnp.float32),
    scratch_types=[
        pltpu.VMEM((GCH, CH), jnp.int32),
        pltpu.VMEM((CH, DW), jnp.float32),
        pltpu.VMEM((CH, DW), jnp.float32),
        pltpu.VMEM_SHARED((N_PAD, DW), jnp.float32),
    ],
    compiler_params=pltpu.CompilerParams(use_tc_tiling_on_sc=False),
)
def _deg_kernel(src_hbm, dst_hbm, ones_hbm, zeros_hbm, out_hbm,
                idx_v, ones_v, zeros_v, acc):
    c = lax.axis_index("c")
    s = lax.axis_index("s")
    wid = s * SC_CORES + c
    row0 = s * STRIDE
    pltpu.sync_copy(ones_hbm, ones_v)
    pltpu.sync_copy(zeros_hbm, zeros_v)
    for io in range(2):
        e_hbm = dst_hbm if io == 0 else src_hbm
        _zero_stripe(zeros_v, acc, row0)
        plsc.subcore_barrier()

        def gbody(g, _):
            pltpu.sync_copy(e_hbm.at[pl.ds(wid * NCHUNK + g * GCH, GCH)],
                            idx_v)

            def ebody(j, _):
                pltpu.sync_copy(ones_v, acc.at[idx_v.at[j]], add=True)
                return 0
            lax.fori_loop(0, GCH, ebody, 0)
            return 0
        lax.fori_loop(0, NGRP, gbody, 0)
        plsc.subcore_barrier()
        pltpu.sync_copy(acc.at[pl.ds(row0, STRIDE)],
                        out_hbm.at[io, c, pl.ds(row0, STRIDE)])


@functools.partial(
    pl.kernel,
    mesh=plsc.VectorSubcoreMesh(**_MESH),
    out_type=jax.ShapeDtypeStruct((NSLAB, SC_CORES, N_PAD, SLABW),
                                  jnp.float32),
    scratch_types=[
        pltpu.VMEM((GCH, CH), jnp.int32),
        pltpu.VMEM((GCH, CH), jnp.int32),
        pltpu.VMEM((NBUF, CH, SLABW), jnp.float32),
        pltpu.VMEM((CH, SLABW), jnp.float32),
        pltpu.VMEM_SHARED((N_PAD, SLABW), jnp.float32),
    ] + [pltpu.SemaphoreType.DMA] * NBUF,
    compiler_params=pltpu.CompilerParams(use_tc_tiling_on_sc=False),
)
def _prop_kernel(tab0_hbm, tab1_hbm, tab2_hbm, tab3_hbm, tab4_hbm, tab5_hbm,
                 src_hbm, dst_hbm, zeros_hbm, out_hbm, sidx_v, didx_v, rows_v,
                 zeros_v, acc, *sems):
    c = lax.axis_index("c")
    s = lax.axis_index("s")
    wid = s * SC_CORES + c
    row0 = s * STRIDE
    pltpu.sync_copy(zeros_hbm, zeros_v)
    for t, tab in enumerate((tab0_hbm, tab1_hbm, tab2_hbm, tab3_hbm,
                             tab4_hbm, tab5_hbm)):
        _zero_stripe(zeros_v, acc, row0)
        plsc.subcore_barrier()

        def gbody(g, _):
            base = wid * NCHUNK + g * GCH
            pltpu.sync_copy(src_hbm.at[pl.ds(base, GCH)], sidx_v)
            pltpu.sync_copy(dst_hbm.at[pl.ds(base, GCH)], didx_v)
            for b in range(4):
                pltpu.async_copy(tab.at[sidx_v.at[b]], rows_v.at[b], sems[b])

            def rbody(jj, _):
                j0 = NBUF * jj
                for b in range(NBUF):
                    j = j0 + b
                    pltpu.make_async_copy(tab.at[sidx_v.at[j]], rows_v.at[b],
                                          sems[b]).wait()
                    pltpu.async_copy(rows_v.at[b], acc.at[didx_v.at[j]],
                                     sems[b], add=True)
                    bn = (b + 4) % NBUF

                    @pl.when(j >= 4)
                    def _():
                        pltpu.make_async_copy(rows_v.at[bn],
                                              acc.at[didx_v.at[j - 4]],
                                              sems[bn]).wait()

                    @pl.when(j + 4 < GCH)
                    def _():
                        pltpu.async_copy(tab.at[sidx_v.at[j + 4]],
                                         rows_v.at[bn], sems[bn])
                return 0
            lax.fori_loop(0, GCH // NBUF, rbody, 0)
            for k in range(4):
                j = GCH - 4 + k
                b = j % NBUF
                pltpu.make_async_copy(rows_v.at[b], acc.at[didx_v.at[j]],
                                      sems[b]).wait()
            return 0
        lax.fori_loop(0, NGRP, gbody, 0)
        plsc.subcore_barrier()
        pltpu.sync_copy(acc.at[pl.ds(row0, STRIDE)],
                        out_hbm.at[t, c, pl.ds(row0, STRIDE)])


def _norm_body(d_ref, o_ref):
    d = d_ref[0, 0] + d_ref[0, 1]
    o_ref[0] = lax.rsqrt(jnp.maximum(d, 1.0))


def _norms(deg):
    nrow = N_PAD * DW // 128
    d = deg.reshape(2, SC_CORES, nrow, 128)
    return pl.pallas_call(
        _norm_body,
        grid=(2, nrow // 128),
        in_specs=[pl.BlockSpec((1, SC_CORES, 128, 128),
                               lambda io, r: (io, 0, r, 0))],
        out_specs=pl.BlockSpec((1, 128, 128), lambda io, r: (io, r, 0)),
        out_shape=jax.ShapeDtypeStruct((2, nrow, 128), jnp.float32),
    )(d).reshape(2, N_PAD, DW)


def _enc_body(x_ref, w_ref, b_ref, n_ref, o_ref):
    y = lax.dot_general(x_ref[...], w_ref[...], (((1,), (0,)), ((), ())),
                        preferred_element_type=jnp.float32)
    o_ref[...] = (y + b_ref[...]) * n_ref[...]


def _encode(feats, w, b, nout16, row_off):
    nt = feats.shape[0]
    bt = 1000
    return pl.pallas_call(
        _enc_body,
        grid=(nt // bt,),
        in_specs=[
            pl.BlockSpec((bt, 128), lambda r: (r, 0)),
            pl.BlockSpec((128, HH), lambda r: (0, 0)),
            pl.BlockSpec((1, HH), lambda r: (0, 0)),
            pl.BlockSpec((bt, HH), lambda r: (r + row_off // bt, 0)),
        ],
        out_specs=pl.BlockSpec((bt, HH), lambda r: (r, 0)),
        out_shape=jax.ShapeDtypeStruct((nt, HH), jnp.float32),
    )(feats, w, b.reshape(1, HH), nout16)


def _combine(p_ref, ni_ref):
    m = jnp.concatenate([p_ref[k, 0] + p_ref[k, 1] for k in range(NSLAB)],
                        axis=1)
    return m * ni_ref[...]


def _stage1_body(p_ref, ni_ref, no_ref, o_ref):
    o_ref[...] = jnp.maximum(_combine(p_ref, ni_ref), 0.0) * no_ref[...]


def _stage2_body(p_ref, ni_ref, no_ref, w_ref, enc_ref, tab_ref):
    m = _combine(p_ref, ni_ref)
    hs = []
    for t in range(NTS):
        hs.append(lax.dot_general(m[:, t * HH:(t + 1) * HH], w_ref[t],
                                  (((1,), (0,)), ((), ())),
                                  preferred_element_type=jnp.float32))
    h = jnp.maximum(jnp.concatenate(hs, axis=1), 0.0)
    enc_ref[...] = h
    tab_ref[...] = h * no_ref[...]


def _stage3_body(p_ref, ni_ref, w_ref, o_ref):
    m = _combine(p_ref, ni_ref)
    acc = jnp.zeros((m.shape[0], CO), jnp.float32)
    for t in range(NTS):
        acc = acc + lax.dot_general(m[:, t * HH:(t + 1) * HH], w_ref[t],
                                    (((1,), (0,)), ((), ())),
                                    preferred_element_type=jnp.float32)
    o_ref[...] = acc * (1.0 / NTS)


_BT = 1024


def _pspec():
    return pl.BlockSpec((NSLAB, SC_CORES, _BT, SLABW), lambda r: (0, 0, r, 0))


def _nspec():
    return pl.BlockSpec((_BT, FW), lambda r: (r, 0))


def _stage1(p, nin48, nout48):
    return pl.pallas_call(
        _stage1_body,
        grid=(N_PAD // _BT,),
        in_specs=[_pspec(), _nspec(), _nspec()],
        out_specs=pl.BlockSpec((_BT, FW), lambda r: (r, 0)),
        out_shape=jax.ShapeDtypeStruct((N_PAD, FW), jnp.float32),
    )(p, nin48, nout48)


def _stage2(p, nin48, nout48, w1):
    return pl.pallas_call(
        _stage2_body,
        grid=(N_PAD // _BT,),
        in_specs=[_pspec(), _nspec(), _nspec(),
                  pl.BlockSpec((NTS, HH, HH), lambda r: (0, 0, 0))],
        out_specs=[pl.BlockSpec((_BT, FW), lambda r: (r, 0)),
                   pl.BlockSpec((_BT, FW), lambda r: (r, 0))],
        out_shape=[jax.ShapeDtypeStruct((N_PAD, FW), jnp.float32),
                   jax.ShapeDtypeStruct((N_PAD, FW), jnp.float32)],
    )(p, nin48, nout48, w1)


def _stage3(p, nin48, w2):
    return pl.pallas_call(
        _stage3_body,
        grid=(N_PAD // _BT,),
        in_specs=[_pspec(), _nspec(),
                  pl.BlockSpec((NTS, HH, CO), lambda r: (0, 0, 0))],
        out_specs=pl.BlockSpec((_BT, CO), lambda r: (r, 0)),
        out_shape=jax.ShapeDtypeStruct((N_PAD, CO), jnp.float32),
    )(p, nin48, w2)


def _slabs(tab48):
    return tuple(tab48[:, k * SLABW:(k + 1) * SLABW] for k in range(NSLAB))


def kernel(features_0, features_1, features_2, e_feat, edge_index,
           W_fc0, b_fc0, W_fc1, b_fc1, W_fc2, b_fc2, W1, W2):
    pad = jnp.full((E_PAD - EE,), NN, jnp.int32)
    src2d = jnp.concatenate([edge_index[0], pad]).reshape(E_PAD // CH, CH)
    dst2d = jnp.concatenate([edge_index[1], pad]).reshape(E_PAD // CH, CH)
    ones_blk = jnp.ones((CH, DW), jnp.float32)
    zeros_blk8 = jnp.zeros((CH, DW), jnp.float32)
    zeros_blk = jnp.zeros((CH, SLABW), jnp.float32)

    deg = _deg_kernel(src2d, dst2d, ones_blk, zeros_blk8)
    norms = _norms(deg)
    nin48 = jnp.broadcast_to(norms[0, :, 0:1], (N_PAD, FW))
    nout48 = jnp.broadcast_to(norms[1, :, 0:1], (N_PAD, FW))
    nout16 = nout48[:, :HH]

    enc0 = _encode(features_0, W_fc0, b_fc0, nout16, 0)
    enc1 = _encode(features_1, W_fc1, b_fc1, nout16, NA)
    enc2 = _encode(features_2, W_fc2, b_fc2, nout16, NA + NB)
    h0 = jnp.zeros((N_PAD, FW), jnp.float32)
    h0 = h0.at[0:NA, 0:HH].set(enc0)
    h0 = h0.at[NA:NA + NB, HH:2 * HH].set(enc1)
    h0 = h0.at[NA + NB:NN, 2 * HH:3 * HH].set(enc2)

    p1 = _prop_kernel(*_slabs(h0), src2d, dst2d, zeros_blk)
    tab1 = _stage1(p1, nin48, nout48)

    p2 = _prop_kernel(*_slabs(tab1), src2d, dst2d, zeros_blk)
    enc48, tab2 = _stage2(p2, nin48, nout48, W1)

    p3 = _prop_kernel(*_slabs(tab2), src2d, dst2d, zeros_blk)
    logits = _stage3(p3, nin48, W2)

    return (logits[:NN], enc48[:NN])


# speedup vs baseline: 8.2688x; 1.0718x over previous
import functools

import jax
import jax.numpy as jnp
from jax import lax
from jax.experimental import pallas as pl
from jax.experimental.pallas import tpu as pltpu
from jax.experimental.pallas import tpu_sc as plsc

NA, NB, NC_T = 40000, 30000, 30000
NN = NA + NB + NC_T
EE = 1600000
NTS = 3
HH = 16
FW = NTS * HH
CO = 8

SC_CORES, SC_SUBCORES = 2, 16
NW = SC_CORES * SC_SUBCORES
CH = 128
NCHUNK = 392
GCH = 56
NGRP = NCHUNK // GCH
NBUF = 14
LEAD = NBUF // 2
EPT = CH * NCHUNK
E_PAD = EPT * NW
N_PAD = 100352
STRIDE = N_PAD // SC_SUBCORES
SLABW = 8
NSLAB = FW // SLABW
DW = 8

_MESH = dict(core_axis_name="c", subcore_axis_name="s")


def _zero_stripe(zeros_v, acc, row0):
    def zbody(i, _):
        pltpu.sync_copy(zeros_v, acc.at[pl.ds(row0 + i * CH, CH)])
        return 0
    lax.fori_loop(0, STRIDE // CH, zbody, 0)


@functools.partial(
    pl.kernel,
    mesh=plsc.VectorSubcoreMesh(**_MESH),
    out_type=jax.ShapeDtypeStruct((2, SC_CORES, N_PAD, DW), jnp.float32),
    scratch_types=[
        pltpu.VMEM((GCH, CH), jnp.int32),
        pltpu.VMEM((CH, DW), jnp.float32),
        pltpu.VMEM((CH, DW), jnp.float32),
        pltpu.VMEM_SHARED((N_PAD, DW), jnp.float32),
    ],
    compiler_params=pltpu.CompilerParams(use_tc_tiling_on_sc=False),
)
def _deg_kernel(src_hbm, dst_hbm, ones_hbm, zeros_hbm, out_hbm,
                idx_v, ones_v, zeros_v, acc):
    c = lax.axis_index("c")
    s = lax.axis_index("s")
    wid = s * SC_CORES + c
    row0 = s * STRIDE
    pltpu.sync_copy(ones_hbm, ones_v)
    pltpu.sync_copy(zeros_hbm, zeros_v)
    for io in range(2):
        e_hbm = dst_hbm if io == 0 else src_hbm
        _zero_stripe(zeros_v, acc, row0)
        plsc.subcore_barrier()

        def gbody(g, _):
            pltpu.sync_copy(e_hbm.at[pl.ds(wid * NCHUNK + g * GCH, GCH)],
                            idx_v)

            def ebody(j, _):
                pltpu.sync_copy(ones_v, acc.at[idx_v.at[j]], add=True)
                return 0
            lax.fori_loop(0, GCH, ebody, 0)
            return 0
        lax.fori_loop(0, NGRP, gbody, 0)
        plsc.subcore_barrier()
        pltpu.sync_copy(acc.at[pl.ds(row0, STRIDE)],
                        out_hbm.at[io, c, pl.ds(row0, STRIDE)])


@functools.partial(
    pl.kernel,
    mesh=plsc.VectorSubcoreMesh(**_MESH),
    out_type=jax.ShapeDtypeStruct((NSLAB, SC_CORES, N_PAD, SLABW),
                                  jnp.float32),
    scratch_types=[
        pltpu.VMEM((GCH, CH), jnp.int32),
        pltpu.VMEM((GCH, CH), jnp.int32),
        pltpu.VMEM((NBUF, CH, SLABW), jnp.float32),
        pltpu.VMEM((CH, SLABW), jnp.float32),
        pltpu.VMEM_SHARED((N_PAD, SLABW), jnp.float32),
    ] + [pltpu.SemaphoreType.DMA] * NBUF,
    compiler_params=pltpu.CompilerParams(use_tc_tiling_on_sc=False),
)
def _prop_kernel(tab0_hbm, tab1_hbm, tab2_hbm, tab3_hbm, tab4_hbm, tab5_hbm,
                 src_hbm, dst_hbm, zeros_hbm, out_hbm, sidx_v, didx_v, rows_v,
                 zeros_v, acc, *sems):
    c = lax.axis_index("c")
    s = lax.axis_index("s")
    wid = s * SC_CORES + c
    row0 = s * STRIDE
    pltpu.sync_copy(zeros_hbm, zeros_v)
    for t, tab in enumerate((tab0_hbm, tab1_hbm, tab2_hbm, tab3_hbm,
                             tab4_hbm, tab5_hbm)):
        _zero_stripe(zeros_v, acc, row0)
        plsc.subcore_barrier()

        def gbody(g, _):
            base = wid * NCHUNK + g * GCH
            pltpu.sync_copy(src_hbm.at[pl.ds(base, GCH)], sidx_v)
            pltpu.sync_copy(dst_hbm.at[pl.ds(base, GCH)], didx_v)
            for b in range(LEAD):
                pltpu.async_copy(tab.at[sidx_v.at[b]], rows_v.at[b], sems[b])

            def rbody(jj, _):
                j0 = NBUF * jj
                for b in range(NBUF):
                    j = j0 + b
                    pltpu.make_async_copy(tab.at[sidx_v.at[j]], rows_v.at[b],
                                          sems[b]).wait()
                    pltpu.async_copy(rows_v.at[b], acc.at[didx_v.at[j]],
                                     sems[b], add=True)
                    bn = (b + LEAD) % NBUF

                    @pl.when(j >= LEAD)
                    def _():
                        pltpu.make_async_copy(rows_v.at[bn],
                                              acc.at[didx_v.at[j - LEAD]],
                                              sems[bn]).wait()

                    @pl.when(j + LEAD < GCH)
                    def _():
                        pltpu.async_copy(tab.at[sidx_v.at[j + LEAD]],
                                         rows_v.at[bn], sems[bn])
                return 0
            lax.fori_loop(0, GCH // NBUF, rbody, 0)
            for k in range(LEAD):
                j = GCH - LEAD + k
                b = j % NBUF
                pltpu.make_async_copy(rows_v.at[b], acc.at[didx_v.at[j]],
                                      sems[b]).wait()
            return 0
        lax.fori_loop(0, NGRP, gbody, 0)
        plsc.subcore_barrier()
        pltpu.sync_copy(acc.at[pl.ds(row0, STRIDE)],
                        out_hbm.at[t, c, pl.ds(row0, STRIDE)])


def _norm_body(d_ref, o_ref):
    d = d_ref[0, 0] + d_ref[0, 1]
    o_ref[0] = lax.rsqrt(jnp.maximum(d, 1.0))


def _norms(deg):
    nrow = N_PAD * DW // 128
    d = deg.reshape(2, SC_CORES, nrow, 128)
    return pl.pallas_call(
        _norm_body,
        grid=(2, nrow // 128),
        in_specs=[pl.BlockSpec((1, SC_CORES, 128, 128),
                               lambda io, r: (io, 0, r, 0))],
        out_specs=pl.BlockSpec((1, 128, 128), lambda io, r: (io, r, 0)),
        out_shape=jax.ShapeDtypeStruct((2, nrow, 128), jnp.float32),
    )(d).reshape(2, N_PAD, DW)


def _enc_body(x_ref, w_ref, b_ref, n_ref, o_ref):
    y = lax.dot_general(x_ref[...], w_ref[...], (((1,), (0,)), ((), ())),
                        preferred_element_type=jnp.float32)
    o_ref[...] = (y + b_ref[...]) * n_ref[...]


def _encode(feats, w, b, nout16, row_off):
    nt = feats.shape[0]
    bt = 1000
    return pl.pallas_call(
        _enc_body,
        grid=(nt // bt,),
        in_specs=[
            pl.BlockSpec((bt, 128), lambda r: (r, 0)),
            pl.BlockSpec((128, HH), lambda r: (0, 0)),
            pl.BlockSpec((1, HH), lambda r: (0, 0)),
            pl.BlockSpec((bt, HH), lambda r: (r + row_off // bt, 0)),
        ],
        out_specs=pl.BlockSpec((bt, HH), lambda r: (r, 0)),
        out_shape=jax.ShapeDtypeStruct((nt, HH), jnp.float32),
    )(feats, w, b.reshape(1, HH), nout16)


def _combine(p_ref, ni_ref):
    m = jnp.concatenate([p_ref[k, 0] + p_ref[k, 1] for k in range(NSLAB)],
                        axis=1)
    return m * ni_ref[...]


def _stage1_body(p_ref, ni_ref, no_ref, o_ref):
    o_ref[...] = jnp.maximum(_combine(p_ref, ni_ref), 0.0) * no_ref[...]


def _stage2_body(p_ref, ni_ref, no_ref, w_ref, enc_ref, tab_ref):
    m = _combine(p_ref, ni_ref)
    hs = []
    for t in range(NTS):
        hs.append(lax.dot_general(m[:, t * HH:(t + 1) * HH], w_ref[t],
                                  (((1,), (0,)), ((), ())),
                                  preferred_element_type=jnp.float32))
    h = jnp.maximum(jnp.concatenate(hs, axis=1), 0.0)
    enc_ref[...] = h
    tab_ref[...] = h * no_ref[...]


def _stage3_body(p_ref, ni_ref, w_ref, o_ref):
    m = _combine(p_ref, ni_ref)
    acc = jnp.zeros((m.shape[0], CO), jnp.float32)
    for t in range(NTS):
        acc = acc + lax.dot_general(m[:, t * HH:(t + 1) * HH], w_ref[t],
                                    (((1,), (0,)), ((), ())),
                                    preferred_element_type=jnp.float32)
    o_ref[...] = acc * (1.0 / NTS)


_BT = 1024


def _pspec():
    return pl.BlockSpec((NSLAB, SC_CORES, _BT, SLABW), lambda r: (0, 0, r, 0))


def _nspec():
    return pl.BlockSpec((_BT, FW), lambda r: (r, 0))


def _stage1(p, nin48, nout48):
    return pl.pallas_call(
        _stage1_body,
        grid=(N_PAD // _BT,),
        in_specs=[_pspec(), _nspec(), _nspec()],
        out_specs=pl.BlockSpec((_BT, FW), lambda r: (r, 0)),
        out_shape=jax.ShapeDtypeStruct((N_PAD, FW), jnp.float32),
    )(p, nin48, nout48)


def _stage2(p, nin48, nout48, w1):
    return pl.pallas_call(
        _stage2_body,
        grid=(N_PAD // _BT,),
        in_specs=[_pspec(), _nspec(), _nspec(),
                  pl.BlockSpec((NTS, HH, HH), lambda r: (0, 0, 0))],
        out_specs=[pl.BlockSpec((_BT, FW), lambda r: (r, 0)),
                   pl.BlockSpec((_BT, FW), lambda r: (r, 0))],
        out_shape=[jax.ShapeDtypeStruct((N_PAD, FW), jnp.float32),
                   jax.ShapeDtypeStruct((N_PAD, FW), jnp.float32)],
    )(p, nin48, nout48, w1)


def _stage3(p, nin48, w2):
    return pl.pallas_call(
        _stage3_body,
        grid=(N_PAD // _BT,),
        in_specs=[_pspec(), _nspec(),
                  pl.BlockSpec((NTS, HH, CO), lambda r: (0, 0, 0))],
        out_specs=pl.BlockSpec((_BT, CO), lambda r: (r, 0)),
        out_shape=jax.ShapeDtypeStruct((N_PAD, CO), jnp.float32),
    )(p, nin48, w2)


def _slabs(tab48):
    return tuple(tab48[:, k * SLABW:(k + 1) * SLABW] for k in range(NSLAB))


def kernel(features_0, features_1, features_2, e_feat, edge_index,
           W_fc0, b_fc0, W_fc1, b_fc1, W_fc2, b_fc2, W1, W2):
    pad = jnp.full((E_PAD - EE,), NN, jnp.int32)
    src2d = jnp.concatenate([edge_index[0], pad]).reshape(E_PAD // CH, CH)
    dst2d = jnp.concatenate([edge_index[1], pad]).reshape(E_PAD // CH, CH)
    ones_blk = jnp.ones((CH, DW), jnp.float32)
    zeros_blk8 = jnp.zeros((CH, DW), jnp.float32)
    zeros_blk = jnp.zeros((CH, SLABW), jnp.float32)

    deg = _deg_kernel(src2d, dst2d, ones_blk, zeros_blk8)
    norms = _norms(deg)
    nin48 = jnp.broadcast_to(norms[0, :, 0:1], (N_PAD, FW))
    nout48 = jnp.broadcast_to(norms[1, :, 0:1], (N_PAD, FW))
    nout16 = nout48[:, :HH]

    enc0 = _encode(features_0, W_fc0, b_fc0, nout16, 0)
    enc1 = _encode(features_1, W_fc1, b_fc1, nout16, NA)
    enc2 = _encode(features_2, W_fc2, b_fc2, nout16, NA + NB)
    h0 = jnp.zeros((N_PAD, FW), jnp.float32)
    h0 = h0.at[0:NA, 0:HH].set(enc0)
    h0 = h0.at[NA:NA + NB, HH:2 * HH].set(enc1)
    h0 = h0.at[NA + NB:NN, 2 * HH:3 * HH].set(enc2)

    p1 = _prop_kernel(*_slabs(h0), src2d, dst2d, zeros_blk)
    tab1 = _stage1(p1, nin48, nout48)

    p2 = _prop_kernel(*_slabs(tab1), src2d, dst2d, zeros_blk)
    enc48, tab2 = _stage2(p2, nin48, nout48, W1)

    p3 = _prop_kernel(*_slabs(tab2), src2d, dst2d, zeros_blk)
    logits = _stage3(p3, nin48, W2)

    return (logits[:NN], enc48[:NN])


# speedup vs baseline: 8.3517x; 1.0100x over previous
import functools

import jax
import jax.numpy as jnp
from jax import lax
from jax.experimental import pallas as pl
from jax.experimental.pallas import tpu as pltpu
from jax.experimental.pallas import tpu_sc as plsc

NA, NB, NC_T = 40000, 30000, 30000
NN = NA + NB + NC_T
EE = 1600000
NTS = 3
HH = 16
FW = NTS * HH
CO = 8

SC_CORES, SC_SUBCORES = 2, 16
NW = SC_CORES * SC_SUBCORES
CH = 128
NCHUNK = 392
GCH = 56
NGRP = NCHUNK // GCH
NBUF = 28
LEAD = NBUF // 2
EPT = CH * NCHUNK
E_PAD = EPT * NW
N_PAD = 100352
STRIDE = N_PAD // SC_SUBCORES
SLABW = 8
NSLAB = FW // SLABW
DW = 8

_MESH = dict(core_axis_name="c", subcore_axis_name="s")


def _zero_stripe(zeros_v, acc, row0):
    def zbody(i, _):
        pltpu.sync_copy(zeros_v, acc.at[pl.ds(row0 + i * CH, CH)])
        return 0
    lax.fori_loop(0, STRIDE // CH, zbody, 0)


@functools.partial(
    pl.kernel,
    mesh=plsc.VectorSubcoreMesh(**_MESH),
    out_type=jax.ShapeDtypeStruct((2, SC_CORES, N_PAD, DW), jnp.float32),
    scratch_types=[
        pltpu.VMEM((GCH, CH), jnp.int32),
        pltpu.VMEM((CH, DW), jnp.float32),
        pltpu.VMEM((CH, DW), jnp.float32),
        pltpu.VMEM_SHARED((N_PAD, DW), jnp.float32),
        pltpu.SemaphoreType.DMA,
    ],
    compiler_params=pltpu.CompilerParams(use_tc_tiling_on_sc=False),
)
def _deg_kernel(src_hbm, dst_hbm, ones_hbm, zeros_hbm, out_hbm,
                idx_v, ones_v, zeros_v, acc, sem):
    c = lax.axis_index("c")
    s = lax.axis_index("s")
    wid = s * SC_CORES + c
    row0 = s * STRIDE
    pltpu.sync_copy(ones_hbm, ones_v)
    pltpu.sync_copy(zeros_hbm, zeros_v)
    for io in range(2):
        e_hbm = dst_hbm if io == 0 else src_hbm
        _zero_stripe(zeros_v, acc, row0)
        plsc.subcore_barrier()

        def gbody(g, _):
            pltpu.sync_copy(e_hbm.at[pl.ds(wid * NCHUNK + g * GCH, GCH)],
                            idx_v)

            def ebody(j, _):
                pltpu.async_copy(ones_v, acc.at[idx_v.at[j]], sem, add=True)
                return 0
            lax.fori_loop(0, GCH, ebody, 0)

            def dbody(j, _):
                pltpu.make_async_copy(ones_v, acc.at[idx_v.at[j]], sem).wait()
                return 0
            lax.fori_loop(0, GCH, dbody, 0)
            return 0
        lax.fori_loop(0, NGRP, gbody, 0)
        plsc.subcore_barrier()
        pltpu.sync_copy(acc.at[pl.ds(row0, STRIDE)],
                        out_hbm.at[io, c, pl.ds(row0, STRIDE)])


@functools.partial(
    pl.kernel,
    mesh=plsc.VectorSubcoreMesh(**_MESH),
    out_type=jax.ShapeDtypeStruct((NSLAB, SC_CORES, N_PAD, SLABW),
                                  jnp.float32),
    scratch_types=[
        pltpu.VMEM((GCH, CH), jnp.int32),
        pltpu.VMEM((GCH, CH), jnp.int32),
        pltpu.VMEM((NBUF, CH, SLABW), jnp.float32),
        pltpu.VMEM((CH, SLABW), jnp.float32),
        pltpu.VMEM_SHARED((N_PAD, SLABW), jnp.float32),
    ] + [pltpu.SemaphoreType.DMA] * NBUF,
    compiler_params=pltpu.CompilerParams(use_tc_tiling_on_sc=False),
)
def _prop_kernel(tab0_hbm, tab1_hbm, tab2_hbm, tab3_hbm, tab4_hbm, tab5_hbm,
                 src_hbm, dst_hbm, zeros_hbm, out_hbm, sidx_v, didx_v, rows_v,
                 zeros_v, acc, *sems):
    c = lax.axis_index("c")
    s = lax.axis_index("s")
    wid = s * SC_CORES + c
    row0 = s * STRIDE
    pltpu.sync_copy(zeros_hbm, zeros_v)
    for t, tab in enumerate((tab0_hbm, tab1_hbm, tab2_hbm, tab3_hbm,
                             tab4_hbm, tab5_hbm)):
        _zero_stripe(zeros_v, acc, row0)
        plsc.subcore_barrier()

        def gbody(g, _):
            base = wid * NCHUNK + g * GCH
            pltpu.sync_copy(src_hbm.at[pl.ds(base, GCH)], sidx_v)
            pltpu.sync_copy(dst_hbm.at[pl.ds(base, GCH)], didx_v)
            for b in range(LEAD):
                pltpu.async_copy(tab.at[sidx_v.at[b]], rows_v.at[b], sems[b])

            def rbody(jj, _):
                j0 = NBUF * jj
                for b in range(NBUF):
                    j = j0 + b
                    pltpu.make_async_copy(tab.at[sidx_v.at[j]], rows_v.at[b],
                                          sems[b]).wait()
                    pltpu.async_copy(rows_v.at[b], acc.at[didx_v.at[j]],
                                     sems[b], add=True)
                    bn = (b + LEAD) % NBUF

                    @pl.when(j >= LEAD)
                    def _():
                        pltpu.make_async_copy(rows_v.at[bn],
                                              acc.at[didx_v.at[j - LEAD]],
                                              sems[bn]).wait()

                    @pl.when(j + LEAD < GCH)
                    def _():
                        pltpu.async_copy(tab.at[sidx_v.at[j + LEAD]],
                                         rows_v.at[bn], sems[bn])
                return 0
            lax.fori_loop(0, GCH // NBUF, rbody, 0)
            for k in range(LEAD):
                j = GCH - LEAD + k
                b = j % NBUF
                pltpu.make_async_copy(rows_v.at[b], acc.at[didx_v.at[j]],
                                      sems[b]).wait()
            return 0
        lax.fori_loop(0, NGRP, gbody, 0)
        plsc.subcore_barrier()
        pltpu.sync_copy(acc.at[pl.ds(row0, STRIDE)],
                        out_hbm.at[t, c, pl.ds(row0, STRIDE)])


def _norm_body(d_ref, o_ref):
    d = d_ref[0, 0] + d_ref[0, 1]
    o_ref[0] = lax.rsqrt(jnp.maximum(d, 1.0))


def _norms(deg):
    nrow = N_PAD * DW // 128
    d = deg.reshape(2, SC_CORES, nrow, 128)
    return pl.pallas_call(
        _norm_body,
        grid=(2, nrow // 128),
        in_specs=[pl.BlockSpec((1, SC_CORES, 128, 128),
                               lambda io, r: (io, 0, r, 0))],
        out_specs=pl.BlockSpec((1, 128, 128), lambda io, r: (io, r, 0)),
        out_shape=jax.ShapeDtypeStruct((2, nrow, 128), jnp.float32),
    )(d).reshape(2, N_PAD, DW)


def _enc_body(x_ref, w_ref, b_ref, n_ref, o_ref):
    y = lax.dot_general(x_ref[...], w_ref[...], (((1,), (0,)), ((), ())),
                        preferred_element_type=jnp.float32)
    o_ref[...] = (y + b_ref[...]) * n_ref[...]


def _encode(feats, w, b, nout16, row_off):
    nt = feats.shape[0]
    bt = 1000
    return pl.pallas_call(
        _enc_body,
        grid=(nt // bt,),
        in_specs=[
            pl.BlockSpec((bt, 128), lambda r: (r, 0)),
            pl.BlockSpec((128, HH), lambda r: (0, 0)),
            pl.BlockSpec((1, HH), lambda r: (0, 0)),
            pl.BlockSpec((bt, HH), lambda r: (r + row_off // bt, 0)),
        ],
        out_specs=pl.BlockSpec((bt, HH), lambda r: (r, 0)),
        out_shape=jax.ShapeDtypeStruct((nt, HH), jnp.float32),
    )(feats, w, b.reshape(1, HH), nout16)


def _combine(p_ref, ni_ref):
    m = jnp.concatenate([p_ref[k, 0] + p_ref[k, 1] for k in range(NSLAB)],
                        axis=1)
    return m * ni_ref[...]


def _stage1_body(p_ref, ni_ref, no_ref, o_ref):
    o_ref[...] = jnp.maximum(_combine(p_ref, ni_ref), 0.0) * no_ref[...]


def _stage2_body(p_ref, ni_ref, no_ref, w_ref, enc_ref, tab_ref):
    m = _combine(p_ref, ni_ref)
    hs = []
    for t in range(NTS):
        hs.append(lax.dot_general(m[:, t * HH:(t + 1) * HH], w_ref[t],
                                  (((1,), (0,)), ((), ())),
                                  preferred_element_type=jnp.float32))
    h = jnp.maximum(jnp.concatenate(hs, axis=1), 0.0)
    enc_ref[...] = h
    tab_ref[...] = h * no_ref[...]


def _stage3_body(p_ref, ni_ref, w_ref, o_ref):
    m = _combine(p_ref, ni_ref)
    acc = jnp.zeros((m.shape[0], CO), jnp.float32)
    for t in range(NTS):
        acc = acc + lax.dot_general(m[:, t * HH:(t + 1) * HH], w_ref[t],
                                    (((1,), (0,)), ((), ())),
                                    preferred_element_type=jnp.float32)
    o_ref[...] = acc * (1.0 / NTS)


_BT = 1024


def _pspec():
    return pl.BlockSpec((NSLAB, SC_CORES, _BT, SLABW), lambda r: (0, 0, r, 0))


def _nspec():
    return pl.BlockSpec((_BT, FW), lambda r: (r, 0))


def _stage1(p, nin48, nout48):
    return pl.pallas_call(
        _stage1_body,
        grid=(N_PAD // _BT,),
        in_specs=[_pspec(), _nspec(), _nspec()],
        out_specs=pl.BlockSpec((_BT, FW), lambda r: (r, 0)),
        out_shape=jax.ShapeDtypeStruct((N_PAD, FW), jnp.float32),
    )(p, nin48, nout48)


def _stage2(p, nin48, nout48, w1):
    return pl.pallas_call(
        _stage2_body,
        grid=(N_PAD // _BT,),
        in_specs=[_pspec(), _nspec(), _nspec(),
                  pl.BlockSpec((NTS, HH, HH), lambda r: (0, 0, 0))],
        out_specs=[pl.BlockSpec((_BT, FW), lambda r: (r, 0)),
                   pl.BlockSpec((_BT, FW), lambda r: (r, 0))],
        out_shape=[jax.ShapeDtypeStruct((N_PAD, FW), jnp.float32),
                   jax.ShapeDtypeStruct((N_PAD, FW), jnp.float32)],
    )(p, nin48, nout48, w1)


def _stage3(p, nin48, w2):
    return pl.pallas_call(
        _stage3_body,
        grid=(N_PAD // _BT,),
        in_specs=[_pspec(), _nspec(),
                  pl.BlockSpec((NTS, HH, CO), lambda r: (0, 0, 0))],
        out_specs=pl.BlockSpec((_BT, CO), lambda r: (r, 0)),
        out_shape=jax.ShapeDtypeStruct((N_PAD, CO), jnp.float32),
    )(p, nin48, w2)


def _slabs(tab48):
    return tuple(tab48[:, k * SLABW:(k + 1) * SLABW] for k in range(NSLAB))


def kernel(features_0, features_1, features_2, e_feat, edge_index,
           W_fc0, b_fc0, W_fc1, b_fc1, W_fc2, b_fc2, W1, W2):
    pad = jnp.full((E_PAD - EE,), NN, jnp.int32)
    src2d = jnp.concatenate([edge_index[0], pad]).reshape(E_PAD // CH, CH)
    dst2d = jnp.concatenate([edge_index[1], pad]).reshape(E_PAD // CH, CH)
    ones_blk = jnp.ones((CH, DW), jnp.float32)
    zeros_blk8 = jnp.zeros((CH, DW), jnp.float32)
    zeros_blk = jnp.zeros((CH, SLABW), jnp.float32)

    deg = _deg_kernel(src2d, dst2d, ones_blk, zeros_blk8)
    norms = _norms(deg)
    nin48 = jnp.broadcast_to(norms[0, :, 0:1], (N_PAD, FW))
    nout48 = jnp.broadcast_to(norms[1, :, 0:1], (N_PAD, FW))
    nout16 = nout48[:, :HH]

    enc0 = _encode(features_0, W_fc0, b_fc0, nout16, 0)
    enc1 = _encode(features_1, W_fc1, b_fc1, nout16, NA)
    enc2 = _encode(features_2, W_fc2, b_fc2, nout16, NA + NB)
    h0 = jnp.zeros((N_PAD, FW), jnp.float32)
    h0 = h0.at[0:NA, 0:HH].set(enc0)
    h0 = h0.at[NA:NA + NB, HH:2 * HH].set(enc1)
    h0 = h0.at[NA + NB:NN, 2 * HH:3 * HH].set(enc2)

    p1 = _prop_kernel(*_slabs(h0), src2d, dst2d, zeros_blk)
    tab1 = _stage1(p1, nin48, nout48)

    p2 = _prop_kernel(*_slabs(tab1), src2d, dst2d, zeros_blk)
    enc48, tab2 = _stage2(p2, nin48, nout48, W1)

    p3 = _prop_kernel(*_slabs(tab2), src2d, dst2d, zeros_blk)
    logits = _stage3(p3, nin48, W2)

    return (logits[:NN], enc48[:NN])
